# baseline (speedup 1.0000x reference)
import functools

import jax
import jax.numpy as jnp
from jax import lax
from jax.experimental import pallas as pl
from jax.experimental.pallas import tpu as pltpu
from jax.experimental.pallas import tpu_sc as plsc

_N = 10000
_B = 16
_NC = 2
_NS = 16
_K = 128
_G = 16
_CHW = 80
_CH = _CHW * _NC * _NS
_EP = _CH * _K
_NACC = 10240
_RT = _NACC // _NS
_ZR = 64
_TN = 400
_NB = _N // _TN
_SPLIT0 = 1280


def _sc_mesh():
    return plsc.VectorSubcoreMesh(
        core_axis_name="c", subcore_axis_name="s",
        num_cores=_NC, num_subcores=_NS)


def _sc_degree(dst2):
    @functools.partial(
        pl.kernel,
        out_type=[jax.ShapeDtypeStruct((_NACC, 16), jnp.float32),
                  jax.ShapeDtypeStruct((_NACC, 16), jnp.float32)],
        mesh=_sc_mesh(),
        scratch_types=[
            pltpu.VMEM((_G, _K), jnp.int32),
            pltpu.VMEM((_K, 16), jnp.float32),
            pltpu.VMEM((_ZR, 16), jnp.float32),
            pltpu.VMEM_SHARED((_NACC, 16), jnp.float32),
        ],
    )
    def deg_k(dst_hbm, out0_hbm, out1_hbm, dst_g, ones_v, zb, acc):
        c = lax.axis_index("c")
        s = lax.axis_index("s")
        w = s * _NC + c

        def fillz(r, carry):
            zb[r, pl.ds(0, 16)] = jnp.zeros((16,), jnp.float32)
            return carry
        lax.fori_loop(0, _ZR, fillz, 0)

        def fillo(r, carry):
            ones_v[r, pl.ds(0, 16)] = jnp.ones((16,), jnp.float32)
            return carry
        lax.fori_loop(0, _K, fillo, 0)

        def zcopy(q, carry):
            pltpu.sync_copy(zb, acc.at[pl.ds(s * _RT + q * _ZR, _ZR)])
            return carry
        lax.fori_loop(0, _RT // _ZR, zcopy, 0)
        plsc.subcore_barrier()

        def group(g, carry):
            pltpu.sync_copy(dst_hbm.at[pl.ds(w * _CHW + g * _G, _G)], dst_g)

            def body(j, carry2):
                pltpu.sync_copy(ones_v, acc.at[dst_g.at[j]], add=True)
                return carry2
            lax.fori_loop(0, _G, body, 0)
            return carry
        lax.fori_loop(0, _CHW // _G, group, 0)

        plsc.subcore_barrier()

        @pl.when(c == 0)
        def _():
            pltpu.sync_copy(acc.at[pl.ds(s * _RT, _RT)],
                            out0_hbm.at[pl.ds(s * _RT, _RT)])

        @pl.when(c == 1)
        def _():
            pltpu.sync_copy(acc.at[pl.ds(s * _RT, _RT)],
                            out1_hbm.at[pl.ds(s * _RT, _RT)])

    return deg_k(dst2)


def _sc_scatter(y2, src_t, dst_t, cps0, cps1):
    cht0 = cps0 // _NS
    cht1 = cps1 // _NS

    @functools.partial(
        pl.kernel,
        out_type=[jax.ShapeDtypeStruct((_NACC, 128), jnp.float32),
                  jax.ShapeDtypeStruct((_NACC, 128), jnp.float32)],
        mesh=_sc_mesh(),
        scratch_types=[
            pltpu.VMEM((_G, _K), jnp.int32),
            pltpu.VMEM((_G, _K), jnp.int32),
            pltpu.VMEM((_K, 128), jnp.float32),
            pltpu.VMEM((_ZR, 128), jnp.float32),
            pltpu.SemaphoreType.DMA,
            pltpu.VMEM_SHARED((_NACC, 128), jnp.float32),
        ],
    )
    def scat_k(y_hbm, src_hbm, dst_hbm, out0_hbm, out1_hbm,
               src_g, dst_g, rows, zb, sem, acc):
        c = lax.axis_index("c")
        s = lax.axis_index("s")

        def fillz(r, carry):
            for q in range(8):
                zb[r, pl.ds(q * 16, 16)] = jnp.zeros((16,), jnp.float32)
            return carry
        lax.fori_loop(0, _ZR, fillz, 0)

        def zcopy(q, carry):
            pltpu.sync_copy(zb, acc.at[pl.ds(s * _RT + q * _ZR, _ZR)])
            return carry
        lax.fori_loop(0, _RT // _ZR, zcopy, 0)
        plsc.subcore_barrier()

        tile_base = jnp.where(c == 0, s * cht0, cps0 + s * cht1)
        groups = jnp.where(c == 0, cht0 // _G, cht1 // _G)

        def group(g, carry):
            base = tile_base + g * _G
            pltpu.sync_copy(src_hbm.at[pl.ds(base, _G)], src_g)
            pltpu.sync_copy(dst_hbm.at[pl.ds(base, _G)], dst_g)

            def body(j, carry2):
                pltpu.async_copy(y_hbm.at[src_g.at[j]], rows, sem).wait()
                pltpu.sync_copy(rows, acc.at[dst_g.at[j]], add=True)
                return carry2
            lax.fori_loop(0, _G, body, 0)
            return carry
        lax.fori_loop(0, groups, group, 0)

        plsc.subcore_barrier()

        @pl.when(c == 0)
        def _():
            pltpu.sync_copy(acc.at[pl.ds(s * _RT, _RT)],
                            out0_hbm.at[pl.ds(s * _RT, _RT)])

        @pl.when(c == 1)
        def _():
            pltpu.sync_copy(acc.at[pl.ds(s * _RT, _RT)],
                            out1_hbm.at[pl.ds(s * _RT, _RT)])

    return scat_k(y2, src_t, dst_t)


def _layer0_body(d0, d1, x, w, y_out, dis_out):
    deg = d0[:, 0:1] + d1[:, 0:1] + 1.0
    dis = lax.rsqrt(deg)
    y = jnp.dot(x[...], w[...], preferred_element_type=jnp.float32) * dis
    y_out[...] = jnp.concatenate(
        [y, jnp.zeros((_TN, 64), jnp.float32)], axis=1)
    dis_out[...] = dis


def _tc_layer0(deg0, deg1, x, w1):
    return pl.pallas_call(
        _layer0_body,
        grid=(_NB,),
        in_specs=[
            pl.BlockSpec((_TN, 16), lambda i: (i, 0)),
            pl.BlockSpec((_TN, 16), lambda i: (i, 0)),
            pl.BlockSpec((_TN, 128), lambda i: (i, 0)),
            pl.BlockSpec((128, 64), lambda i: (0, 0)),
        ],
        out_specs=[
            pl.BlockSpec((_TN, 128), lambda i: (i, 0)),
            pl.BlockSpec((_TN, 1), lambda i: (i, 0)),
        ],
        out_shape=[
            jax.ShapeDtypeStruct((_N, 128), jnp.float32),
            jax.ShapeDtypeStruct((_N, 1), jnp.float32),
        ],
    )(deg0, deg1, x, w1)


def _mid1_body(a0, a1, y, dis, b, w, out):
    pre = (a0[...] + a1[...] + y[...])[:, 0:64]
    dis_v = dis[...]
    h = jnp.maximum(pre * dis_v + b[...], 0.0)
    out[...] = jnp.dot(h, w[...], preferred_element_type=jnp.float32) * dis_v


def _tc_mid1(agg0, agg1, y1p, dis, b1, w2):
    return pl.pallas_call(
        _mid1_body,
        grid=(_NB,),
        in_specs=[
            pl.BlockSpec((_TN, 128), lambda i: (i, 0)),
            pl.BlockSpec((_TN, 128), lambda i: (i, 0)),
            pl.BlockSpec((_TN, 128), lambda i: (i, 0)),
            pl.BlockSpec((_TN, 1), lambda i: (i, 0)),
            pl.BlockSpec((1, 64), lambda i: (0, 0)),
            pl.BlockSpec((64, 128), lambda i: (0, 0)),
        ],
        out_specs=pl.BlockSpec((_TN, 128), lambda i: (i, 0)),
        out_shape=jax.ShapeDtypeStruct((_N, 128), jnp.float32),
    )(agg0, agg1, y1p, dis, b1, w2)


def _mid2_body(a0, a1, y, dis, b, w, out):
    dis_v = dis[...]
    h = jnp.maximum((a0[...] + a1[...] + y[...]) * dis_v + b[...], 0.0)
    out[...] = jnp.dot(h, w[...][0],
                       preferred_element_type=jnp.float32) * dis_v


def _tc_mid2(agg0, agg1, y2p, dis, b2, w3r):
    return pl.pallas_call(
        _mid2_body,
        grid=(_NB, 2),
        in_specs=[
            pl.BlockSpec((_TN, 128), lambda i, c: (i, 0)),
            pl.BlockSpec((_TN, 128), lambda i, c: (i, 0)),
            pl.BlockSpec((_TN, 128), lambda i, c: (i, 0)),
            pl.BlockSpec((_TN, 1), lambda i, c: (i, 0)),
            pl.BlockSpec((1, 128), lambda i, c: (0, 0)),
            pl.BlockSpec((1, 128, 128), lambda i, c: (c, 0, 0)),
        ],
        out_specs=pl.BlockSpec((_TN, 128), lambda i, c: (c * _NB + i, 0)),
        out_shape=jax.ShapeDtypeStruct((2 * _N, 128), jnp.float32),
    )(agg0, agg1, y2p, dis, b2, w3r)


def _pool_body(a0, a1, a2, a3, y0, y1, dis, b, batch, psum, pcnt):
    i = pl.program_id(0)
    pre = jnp.concatenate([a0[...] + a1[...] + y0[...],
                           a2[...] + a3[...] + y1[...]], axis=1)
    h = jnp.maximum(pre * dis[...] + b[...], 0.0)
    oh = (batch[...] == lax.broadcasted_iota(jnp.int32, (_TN, _B), 1))
    oh = oh.astype(jnp.float32)

    @pl.when(i == 0)
    def _():
        psum[...] = jnp.zeros_like(psum)
        pcnt[...] = jnp.zeros_like(pcnt)

    dn = (((0,), (0,)), ((), ()))
    psum[...] += lax.dot_general(oh, h, dn,
                                 preferred_element_type=jnp.float32)
    pcnt[...] += lax.dot_general(oh, jnp.ones((_TN, 128), jnp.float32), dn,
                                 preferred_element_type=jnp.float32)


def _tc_pool(agg0, agg1, agg2, agg3, y3, dis, b, batch2):
    return pl.pallas_call(
        _pool_body,
        grid=(_NB,),
        in_specs=[
            pl.BlockSpec((_TN, 128), lambda i: (i, 0)),
            pl.BlockSpec((_TN, 128), lambda i: (i, 0)),
            pl.BlockSpec((_TN, 128), lambda i: (i, 0)),
            pl.BlockSpec((_TN, 128), lambda i: (i, 0)),
            pl.BlockSpec((_TN, 128), lambda i: (i, 0)),
            pl.BlockSpec((_TN, 128), lambda i: (_NB + i, 0)),
            pl.BlockSpec((_TN, 1), lambda i: (i, 0)),
            pl.BlockSpec((1, 256), lambda i: (0, 0)),
            pl.BlockSpec((_TN, 1), lambda i: (i, 0)),
        ],
        out_specs=[
            pl.BlockSpec((_B, 256), lambda i: (0, 0)),
            pl.BlockSpec((_B, 128), lambda i: (0, 0)),
        ],
        out_shape=[
            jax.ShapeDtypeStruct((_B, 256), jnp.float32),
            jax.ShapeDtypeStruct((_B, 128), jnp.float32),
        ],
    )(agg0, agg1, agg2, agg3, y3, y3, dis, b, batch2)


def _sig(x):
    return 0.5 * (jnp.tanh(0.5 * x) + 1.0)


def _head_body(psum, pcnt, wf, bif, bhf, wr, bir, bhr, wfc, bfc, out):
    cnt = jnp.maximum(pcnt[:, 0:1], 1.0)
    pooled = psum[...] / cnt

    def cell(wt, bi, bh):
        g = jnp.dot(pooled, wt[...],
                    preferred_element_type=jnp.float32) + bi[...] + bh[...]
        ii = _sig(g[:, 0:128])
        gg = jnp.tanh(g[:, 256:384])
        oo = _sig(g[:, 384:512])
        return oo * jnp.tanh(ii * gg)

    hf = cell(wf, bif, bhf)
    hr = cell(wr, bir, bhr)
    last = jnp.concatenate([hf, hr], axis=1)
    logits = jnp.dot(last, wfc[...],
                     preferred_element_type=jnp.float32) + bfc[...]
    m = jnp.max(logits, axis=1, keepdims=True)
    z = logits - m
    out[...] = z - jnp.log(jnp.sum(jnp.exp(z), axis=1, keepdims=True))


def _tc_head(psum, pcnt, wf, bif, bhf, wr, bir, bhr, wfc, bfc):
    return pl.pallas_call(
        _head_body,
        out_shape=jax.ShapeDtypeStruct((_B, 500), jnp.float32),
    )(psum, pcnt, wf, bif, bhf, wr, bir, bhr, wfc, bfc)


def kernel(x, edge_index, batch, W1, b1, W2, b2, W3, b3,
           Wih_f, Whh_f, bih_f, bhh_f, Wih_r, Whh_r, bih_r, bhh_r,
           Wfc, bfc):
    src = edge_index[0].astype(jnp.int32)
    dst = edge_index[1].astype(jnp.int32)
    pad = _EP - src.shape[0]
    srcp = jnp.concatenate([src, jnp.zeros((pad,), jnp.int32)])
    dstp = jnp.concatenate([dst, jnp.full((pad,), _N, jnp.int32)])
    src_e = srcp.reshape(_CH, _K)
    dst_e = dstp.reshape(_CH, _K)
    src_e2 = src_e + _N
    batch2 = batch.astype(jnp.int32).reshape(_N, 1)

    deg0, deg1 = _sc_degree(dst_e)
    y1p, dis = _tc_layer0(deg0, deg1, x, W1)
    agg1a, agg1b = _sc_scatter(y1p, src_e, dst_e, _SPLIT0, _CH - _SPLIT0)
    y2p = _tc_mid1(agg1a, agg1b, y1p, dis, b1.reshape(1, -1), W2)
    agg2a, agg2b = _sc_scatter(y2p, src_e, dst_e, _SPLIT0, _CH - _SPLIT0)
    w3r = W3.reshape(128, 2, 128).transpose(1, 0, 2)
    y3 = _tc_mid2(agg2a, agg2b, y2p, dis, b2.reshape(1, -1), w3r)
    agg3a, agg3b = _sc_scatter(y3, src_e, dst_e, _SPLIT0, _CH - _SPLIT0)
    agg3c, agg3d = _sc_scatter(y3, src_e2, dst_e, _SPLIT0, _CH - _SPLIT0)
    psum, pcnt = _tc_pool(agg3a, agg3b, agg3c, agg3d, y3, dis,
                          b3.reshape(1, -1), batch2)
    return _tc_head(psum, pcnt,
                    Wih_f.T, bih_f.reshape(1, -1), bhh_f.reshape(1, -1),
                    Wih_r.T, bih_r.reshape(1, -1), bhh_r.reshape(1, -1),
                    Wfc.T, bfc.reshape(1, -1))


# speedup vs baseline: 1.3034x; 1.3034x over previous
import functools

import jax
import jax.numpy as jnp
from jax import lax
from jax.experimental import pallas as pl
from jax.experimental.pallas import tpu as pltpu
from jax.experimental.pallas import tpu_sc as plsc

_N = 10000
_B = 16
_NC = 2
_NS = 16
_K = 128
_G = 40
_CHW = 80
_CH = _CHW * _NC * _NS
_EP = _CH * _K
_NACC = 10240
_RT = _NACC // _NS
_ZR = 64
_TN = 400
_NB = _N // _TN
_SPLIT0 = 1280


def _sc_mesh():
    return plsc.VectorSubcoreMesh(
        core_axis_name="c", subcore_axis_name="s",
        num_cores=_NC, num_subcores=_NS)


def _sc_degree(dst2):
    @functools.partial(
        pl.kernel,
        out_type=[jax.ShapeDtypeStruct((_NACC, 16), jnp.float32),
                  jax.ShapeDtypeStruct((_NACC, 16), jnp.float32)],
        mesh=_sc_mesh(),
        scratch_types=[
            pltpu.VMEM((_G, _K), jnp.int32),
            pltpu.VMEM((_K, 16), jnp.float32),
            pltpu.VMEM((_ZR, 16), jnp.float32),
            pltpu.VMEM_SHARED((_NACC, 16), jnp.float32),
        ],
    )
    def deg_k(dst_hbm, out0_hbm, out1_hbm, dst_g, ones_v, zb, acc):
        c = lax.axis_index("c")
        s = lax.axis_index("s")
        w = s * _NC + c

        def fillz(r, carry):
            zb[r, pl.ds(0, 16)] = jnp.zeros((16,), jnp.float32)
            return carry
        lax.fori_loop(0, _ZR, fillz, 0)

        def fillo(r, carry):
            ones_v[r, pl.ds(0, 16)] = jnp.ones((16,), jnp.float32)
            return carry
        lax.fori_loop(0, _K, fillo, 0)

        def zcopy(q, carry):
            pltpu.sync_copy(zb, acc.at[pl.ds(s * _RT + q * _ZR, _ZR)])
            return carry
        lax.fori_loop(0, _RT // _ZR, zcopy, 0)
        plsc.subcore_barrier()

        def group(g, carry):
            pltpu.sync_copy(dst_hbm.at[pl.ds(w * _CHW + g * _G, _G)], dst_g)

            def body(j, carry2):
                pltpu.sync_copy(ones_v, acc.at[dst_g.at[j]], add=True)
                return carry2
            lax.fori_loop(0, _G, body, 0)
            return carry
        lax.fori_loop(0, _CHW // _G, group, 0)

        plsc.subcore_barrier()

        @pl.when(c == 0)
        def _():
            pltpu.sync_copy(acc.at[pl.ds(s * _RT, _RT)],
                            out0_hbm.at[pl.ds(s * _RT, _RT)])

        @pl.when(c == 1)
        def _():
            pltpu.sync_copy(acc.at[pl.ds(s * _RT, _RT)],
                            out1_hbm.at[pl.ds(s * _RT, _RT)])

    return deg_k(dst2)


def _sc_scatter(y2, src_t, dst_t, cps0, cps1):
    cht0 = cps0 // _NS
    cht1 = cps1 // _NS

    @functools.partial(
        pl.kernel,
        out_type=[jax.ShapeDtypeStruct((_NACC, 128), jnp.float32),
                  jax.ShapeDtypeStruct((_NACC, 128), jnp.float32)],
        mesh=_sc_mesh(),
        scratch_types=[
            pltpu.VMEM((_G, _K), jnp.int32),
            pltpu.VMEM((_G, _K), jnp.int32),
            pltpu.VMEM((_K, 128), jnp.float32),
            pltpu.VMEM((_ZR, 128), jnp.float32),
            pltpu.SemaphoreType.DMA,
            pltpu.VMEM_SHARED((_NACC, 128), jnp.float32),
        ],
    )
    def scat_k(y_hbm, src_hbm, dst_hbm, out0_hbm, out1_hbm,
               src_g, dst_g, rows, zb, sem, acc):
        c = lax.axis_index("c")
        s = lax.axis_index("s")

        def fillz(r, carry):
            for q in range(8):
                zb[r, pl.ds(q * 16, 16)] = jnp.zeros((16,), jnp.float32)
            return carry
        lax.fori_loop(0, _ZR, fillz, 0)

        def zcopy(q, carry):
            pltpu.sync_copy(zb, acc.at[pl.ds(s * _RT + q * _ZR, _ZR)])
            return carry
        lax.fori_loop(0, _RT // _ZR, zcopy, 0)
        plsc.subcore_barrier()

        tile_base = jnp.where(c == 0, s * cht0, cps0 + s * cht1)
        groups = jnp.where(c == 0, cht0 // _G, cht1 // _G)

        def group(g, carry):
            base = tile_base + g * _G
            pltpu.sync_copy(src_hbm.at[pl.ds(base, _G)], src_g)
            pltpu.sync_copy(dst_hbm.at[pl.ds(base, _G)], dst_g)

            def body(j, carry2):
                pltpu.async_copy(y_hbm.at[src_g.at[j]], rows, sem).wait()
                pltpu.sync_copy(rows, acc.at[dst_g.at[j]], add=True)
                return carry2
            lax.fori_loop(0, _G, body, 0)
            return carry
        lax.fori_loop(0, groups, group, 0)

        plsc.subcore_barrier()

        @pl.when(c == 0)
        def _():
            pltpu.sync_copy(acc.at[pl.ds(s * _RT, _RT)],
                            out0_hbm.at[pl.ds(s * _RT, _RT)])

        @pl.when(c == 1)
        def _():
            pltpu.sync_copy(acc.at[pl.ds(s * _RT, _RT)],
                            out1_hbm.at[pl.ds(s * _RT, _RT)])

    return scat_k(y2, src_t, dst_t)


def _layer0_body(d0, d1, x, w, y_out, dis_out):
    deg = d0[:, 0:1] + d1[:, 0:1] + 1.0
    dis = lax.rsqrt(deg)
    y = jnp.dot(x[...], w[...], preferred_element_type=jnp.float32) * dis
    y_out[...] = jnp.concatenate(
        [y, jnp.zeros((_TN, 64), jnp.float32)], axis=1)
    dis_out[...] = dis


def _tc_layer0(deg0, deg1, x, w1):
    return pl.pallas_call(
        _layer0_body,
        grid=(_NB,),
        in_specs=[
            pl.BlockSpec((_TN, 16), lambda i: (i, 0)),
            pl.BlockSpec((_TN, 16), lambda i: (i, 0)),
            pl.BlockSpec((_TN, 128), lambda i: (i, 0)),
            pl.BlockSpec((128, 64), lambda i: (0, 0)),
        ],
        out_specs=[
            pl.BlockSpec((_TN, 128), lambda i: (i, 0)),
            pl.BlockSpec((_TN, 1), lambda i: (i, 0)),
        ],
        out_shape=[
            jax.ShapeDtypeStruct((_N, 128), jnp.float32),
            jax.ShapeDtypeStruct((_N, 1), jnp.float32),
        ],
    )(deg0, deg1, x, w1)


def _mid1_body(a0, a1, y, dis, b, w, out):
    pre = (a0[...] + a1[...] + y[...])[:, 0:64]
    dis_v = dis[...]
    h = jnp.maximum(pre * dis_v + b[...], 0.0)
    out[...] = jnp.dot(h, w[...], preferred_element_type=jnp.float32) * dis_v


def _tc_mid1(agg0, agg1, y1p, dis, b1, w2):
    return pl.pallas_call(
        _mid1_body,
        grid=(_NB,),
        in_specs=[
            pl.BlockSpec((_TN, 128), lambda i: (i, 0)),
            pl.BlockSpec((_TN, 128), lambda i: (i, 0)),
            pl.BlockSpec((_TN, 128), lambda i: (i, 0)),
            pl.BlockSpec((_TN, 1), lambda i: (i, 0)),
            pl.BlockSpec((1, 64), lambda i: (0, 0)),
            pl.BlockSpec((64, 128), lambda i: (0, 0)),
        ],
        out_specs=pl.BlockSpec((_TN, 128), lambda i: (i, 0)),
        out_shape=jax.ShapeDtypeStruct((_N, 128), jnp.float32),
    )(agg0, agg1, y1p, dis, b1, w2)


def _mid2_body(a0, a1, y, dis, b, w, out):
    dis_v = dis[...]
    h = jnp.maximum((a0[...] + a1[...] + y[...]) * dis_v + b[...], 0.0)
    out[...] = jnp.dot(h, w[...][0],
                       preferred_element_type=jnp.float32) * dis_v


def _tc_mid2(agg0, agg1, y2p, dis, b2, w3r):
    return pl.pallas_call(
        _mid2_body,
        grid=(_NB, 2),
        in_specs=[
            pl.BlockSpec((_TN, 128), lambda i, c: (i, 0)),
            pl.BlockSpec((_TN, 128), lambda i, c: (i, 0)),
            pl.BlockSpec((_TN, 128), lambda i, c: (i, 0)),
            pl.BlockSpec((_TN, 1), lambda i, c: (i, 0)),
            pl.BlockSpec((1, 128), lambda i, c: (0, 0)),
            pl.BlockSpec((1, 128, 128), lambda i, c: (c, 0, 0)),
        ],
        out_specs=pl.BlockSpec((_TN, 128), lambda i, c: (c * _NB + i, 0)),
        out_shape=jax.ShapeDtypeStruct((2 * _N, 128), jnp.float32),
    )(agg0, agg1, y2p, dis, b2, w3r)


def _pool_body(a0, a1, y0, y1, dis, b, batch, psum, pcnt):
    i = pl.program_id(0)
    pre = jnp.concatenate([a0[...] + y0[...], a1[...] + y1[...]], axis=1)
    h = jnp.maximum(pre * dis[...] + b[...], 0.0)
    oh = (batch[...] == lax.broadcasted_iota(jnp.int32, (_TN, _B), 1))
    oh = oh.astype(jnp.float32)

    @pl.when(i == 0)
    def _():
        psum[...] = jnp.zeros_like(psum)
        pcnt[...] = jnp.zeros_like(pcnt)

    dn = (((0,), (0,)), ((), ()))
    psum[...] += lax.dot_general(oh, h, dn,
                                 preferred_element_type=jnp.float32)
    pcnt[...] += lax.dot_general(oh, jnp.ones((_TN, 128), jnp.float32), dn,
                                 preferred_element_type=jnp.float32)


def _tc_pool(agg0, agg1, y3, dis, b, batch2):
    return pl.pallas_call(
        _pool_body,
        grid=(_NB,),
        in_specs=[
            pl.BlockSpec((_TN, 128), lambda i: (i, 0)),
            pl.BlockSpec((_TN, 128), lambda i: (i, 0)),
            pl.BlockSpec((_TN, 128), lambda i: (i, 0)),
            pl.BlockSpec((_TN, 128), lambda i: (_NB + i, 0)),
            pl.BlockSpec((_TN, 1), lambda i: (i, 0)),
            pl.BlockSpec((1, 256), lambda i: (0, 0)),
            pl.BlockSpec((_TN, 1), lambda i: (i, 0)),
        ],
        out_specs=[
            pl.BlockSpec((_B, 256), lambda i: (0, 0)),
            pl.BlockSpec((_B, 128), lambda i: (0, 0)),
        ],
        out_shape=[
            jax.ShapeDtypeStruct((_B, 256), jnp.float32),
            jax.ShapeDtypeStruct((_B, 128), jnp.float32),
        ],
    )(agg0, agg1, y3, y3, dis, b, batch2)


def _sig(x):
    return 0.5 * (jnp.tanh(0.5 * x) + 1.0)


def _head_body(psum, pcnt, wf, bif, bhf, wr, bir, bhr, wfc, bfc, out):
    cnt = jnp.maximum(pcnt[:, 0:1], 1.0)
    pooled = psum[...] / cnt

    def cell(wt, bi, bh):
        g = jnp.dot(pooled, wt[...],
                    preferred_element_type=jnp.float32) + bi[...] + bh[...]
        ii = _sig(g[:, 0:128])
        gg = jnp.tanh(g[:, 256:384])
        oo = _sig(g[:, 384:512])
        return oo * jnp.tanh(ii * gg)

    hf = cell(wf, bif, bhf)
    hr = cell(wr, bir, bhr)
    last = jnp.concatenate([hf, hr], axis=1)
    logits = jnp.dot(last, wfc[...],
                     preferred_element_type=jnp.float32) + bfc[...]
    m = jnp.max(logits, axis=1, keepdims=True)
    z = logits - m
    out[...] = z - jnp.log(jnp.sum(jnp.exp(z), axis=1, keepdims=True))


def _tc_head(psum, pcnt, wf, bif, bhf, wr, bir, bhr, wfc, bfc):
    return pl.pallas_call(
        _head_body,
        out_shape=jax.ShapeDtypeStruct((_B, 500), jnp.float32),
    )(psum, pcnt, wf, bif, bhf, wr, bir, bhr, wfc, bfc)


def kernel(x, edge_index, batch, W1, b1, W2, b2, W3, b3,
           Wih_f, Whh_f, bih_f, bhh_f, Wih_r, Whh_r, bih_r, bhh_r,
           Wfc, bfc):
    src = edge_index[0].astype(jnp.int32)
    dst = edge_index[1].astype(jnp.int32)
    pad = _EP - src.shape[0]
    srcp = jnp.concatenate([src, jnp.zeros((pad,), jnp.int32)])
    dstp = jnp.concatenate([dst, jnp.full((pad,), _N, jnp.int32)])
    src_e = srcp.reshape(_CH, _K)
    dst_e = dstp.reshape(_CH, _K)
    src_f = jnp.concatenate([srcp, srcp + _N]).reshape(2 * _CH, _K)
    dst_f = jnp.concatenate([dst_e, dst_e])
    batch2 = batch.astype(jnp.int32).reshape(_N, 1)

    deg0, deg1 = _sc_degree(dst_e)
    y1p, dis = _tc_layer0(deg0, deg1, x, W1)
    agg1a, agg1b = _sc_scatter(y1p, src_e, dst_e, _SPLIT0, _CH - _SPLIT0)
    y2p = _tc_mid1(agg1a, agg1b, y1p, dis, b1.reshape(1, -1), W2)
    agg2a, agg2b = _sc_scatter(y2p, src_e, dst_e, _SPLIT0, _CH - _SPLIT0)
    w3r = W3.reshape(128, 2, 128).transpose(1, 0, 2)
    y3 = _tc_mid2(agg2a, agg2b, y2p, dis, b2.reshape(1, -1), w3r)
    agg3a, agg3b = _sc_scatter(y3, src_f, dst_f, _CH, _CH)
    psum, pcnt = _tc_pool(agg3a, agg3b, y3, dis, b3.reshape(1, -1), batch2)
    return _tc_head(psum, pcnt,
                    Wih_f.T, bih_f.reshape(1, -1), bhh_f.reshape(1, -1),
                    Wih_r.T, bih_r.reshape(1, -1), bhh_r.reshape(1, -1),
                    Wfc.T, bfc.reshape(1, -1))


# speedup vs baseline: 2.8152x; 2.1598x over previous
import functools

import jax
import jax.numpy as jnp
from jax import lax
from jax.experimental import pallas as pl
from jax.experimental.pallas import tpu as pltpu
from jax.experimental.pallas import tpu_sc as plsc

_N = 10000
_B = 16
_NC = 2
_NS = 16
_K = 128
_G = 40
_CHW = 80
_CH = _CHW * _NC * _NS
_EP = _CH * _K
_NACC = 10240
_RT = _NACC // _NS
_ZR = 64
_TN = 400
_NB = _N // _TN
_SPLIT0 = 1280


def _sc_mesh():
    return plsc.VectorSubcoreMesh(
        core_axis_name="c", subcore_axis_name="s",
        num_cores=_NC, num_subcores=_NS)


def _sc_degree(dst2):
    @functools.partial(
        pl.kernel,
        out_type=[jax.ShapeDtypeStruct((_NACC, 16), jnp.float32),
                  jax.ShapeDtypeStruct((_NACC, 16), jnp.float32)],
        mesh=_sc_mesh(),
        scratch_types=[
            pltpu.VMEM((_G, _K), jnp.int32),
            pltpu.VMEM((_K, 16), jnp.float32),
            pltpu.VMEM((_ZR, 16), jnp.float32),
            pltpu.VMEM_SHARED((_NACC, 16), jnp.float32),
        ],
    )
    def deg_k(dst_hbm, out0_hbm, out1_hbm, dst_g, ones_v, zb, acc):
        c = lax.axis_index("c")
        s = lax.axis_index("s")
        w = s * _NC + c

        def fillz(r, carry):
            zb[r, pl.ds(0, 16)] = jnp.zeros((16,), jnp.float32)
            return carry
        lax.fori_loop(0, _ZR, fillz, 0)

        def fillo(r, carry):
            ones_v[r, pl.ds(0, 16)] = jnp.ones((16,), jnp.float32)
            return carry
        lax.fori_loop(0, _K, fillo, 0)

        def zcopy(q, carry):
            pltpu.sync_copy(zb, acc.at[pl.ds(s * _RT + q * _ZR, _ZR)])
            return carry
        lax.fori_loop(0, _RT // _ZR, zcopy, 0)
        plsc.subcore_barrier()

        def group(g, carry):
            pltpu.sync_copy(dst_hbm.at[pl.ds(w * _CHW + g * _G, _G)], dst_g)

            def body(j, carry2):
                pltpu.sync_copy(ones_v, acc.at[dst_g.at[j]], add=True)
                return carry2
            lax.fori_loop(0, _G, body, 0)
            return carry
        lax.fori_loop(0, _CHW // _G, group, 0)

        plsc.subcore_barrier()

        @pl.when(c == 0)
        def _():
            pltpu.sync_copy(acc.at[pl.ds(s * _RT, _RT)],
                            out0_hbm.at[pl.ds(s * _RT, _RT)])

        @pl.when(c == 1)
        def _():
            pltpu.sync_copy(acc.at[pl.ds(s * _RT, _RT)],
                            out1_hbm.at[pl.ds(s * _RT, _RT)])

    return deg_k(dst2)


def _sc_scatter(y2, src_t, dst_t, cps0, cps1):
    cht0 = cps0 // _NS
    cht1 = cps1 // _NS

    @functools.partial(
        pl.kernel,
        out_type=[jax.ShapeDtypeStruct((_NACC, 128), jnp.float32),
                  jax.ShapeDtypeStruct((_NACC, 128), jnp.float32)],
        mesh=_sc_mesh(),
        scratch_types=[
            pltpu.VMEM((_G, _K), jnp.int32),
            pltpu.VMEM((_G, _K), jnp.int32),
            pltpu.VMEM((_K, 128), jnp.float32),
            pltpu.VMEM((_ZR, 128), jnp.float32),
            pltpu.SemaphoreType.DMA,
            pltpu.VMEM_SHARED((_NACC, 128), jnp.float32),
        ],
    )
    def scat_k(y_hbm, src_hbm, dst_hbm, out0_hbm, out1_hbm,
               src_g, dst_g, rows, zb, sem, acc):
        c = lax.axis_index("c")
        s = lax.axis_index("s")

        def fillz(r, carry):
            for q in range(8):
                zb[r, pl.ds(q * 16, 16)] = jnp.zeros((16,), jnp.float32)
            return carry
        lax.fori_loop(0, _ZR, fillz, 0)

        def zcopy(q, carry):
            pltpu.sync_copy(zb, acc.at[pl.ds(s * _RT + q * _ZR, _ZR)])
            return carry
        lax.fori_loop(0, _RT // _ZR, zcopy, 0)
        plsc.subcore_barrier()

        tile_base = jnp.where(c == 0, s * cht0, cps0 + s * cht1)
        groups = jnp.where(c == 0, cht0 // _G, cht1 // _G)

        def group(g, carry):
            base = tile_base + g * _G
            pltpu.sync_copy(src_hbm.at[pl.ds(base, _G)], src_g)
            pltpu.sync_copy(dst_hbm.at[pl.ds(base, _G)], dst_g)

            def body(j, carry2):
                pltpu.async_copy(y_hbm.at[src_g.at[j]], rows, sem).wait()
                pltpu.sync_copy(rows, acc.at[dst_g.at[j]], add=True)
                return carry2
            lax.fori_loop(0, _G, body, 0)
            return carry
        lax.fori_loop(0, groups, group, 0)

        plsc.subcore_barrier()

        @pl.when(c == 0)
        def _():
            pltpu.sync_copy(acc.at[pl.ds(s * _RT, _RT)],
                            out0_hbm.at[pl.ds(s * _RT, _RT)])

        @pl.when(c == 1)
        def _():
            pltpu.sync_copy(acc.at[pl.ds(s * _RT, _RT)],
                            out1_hbm.at[pl.ds(s * _RT, _RT)])

    return scat_k(y2, src_t, dst_t)


def _layer0_body(d0, d1, x, w, y_out, dis_out):
    deg = d0[:, 0:1] + d1[:, 0:1] + 1.0
    dis = lax.rsqrt(deg)
    y = jnp.dot(x[...], w[...], preferred_element_type=jnp.float32) * dis
    y_out[...] = jnp.concatenate(
        [y, jnp.zeros((_TN, 64), jnp.float32)], axis=1)
    dis_out[...] = dis


def _tc_layer0(deg0, deg1, x, w1):
    return pl.pallas_call(
        _layer0_body,
        grid=(_NB,),
        in_specs=[
            pl.BlockSpec((_TN, 16), lambda i: (i, 0)),
            pl.BlockSpec((_TN, 16), lambda i: (i, 0)),
            pl.BlockSpec((_TN, 128), lambda i: (i, 0)),
            pl.BlockSpec((128, 64), lambda i: (0, 0)),
        ],
        out_specs=[
            pl.BlockSpec((_TN, 128), lambda i: (i, 0)),
            pl.BlockSpec((_TN, 1), lambda i: (i, 0)),
        ],
        out_shape=[
            jax.ShapeDtypeStruct((_N, 128), jnp.float32),
            jax.ShapeDtypeStruct((_N, 1), jnp.float32),
        ],
    )(deg0, deg1, x, w1)


def _mid1_body(a0, a1, y, dis, b, w, out):
    pre = (a0[...] + a1[...] + y[...])[:, 0:64]
    dis_v = dis[...]
    h = jnp.maximum(pre * dis_v + b[...], 0.0)
    out[...] = jnp.dot(h, w[...], preferred_element_type=jnp.float32) * dis_v


def _tc_mid1(agg0, agg1, y1p, dis, b1, w2):
    return pl.pallas_call(
        _mid1_body,
        grid=(_NB,),
        in_specs=[
            pl.BlockSpec((_TN, 128), lambda i: (i, 0)),
            pl.BlockSpec((_TN, 128), lambda i: (i, 0)),
            pl.BlockSpec((_TN, 128), lambda i: (i, 0)),
            pl.BlockSpec((_TN, 1), lambda i: (i, 0)),
            pl.BlockSpec((1, 64), lambda i: (0, 0)),
            pl.BlockSpec((64, 128), lambda i: (0, 0)),
        ],
        out_specs=pl.BlockSpec((_TN, 128), lambda i: (i, 0)),
        out_shape=jax.ShapeDtypeStruct((_N, 128), jnp.float32),
    )(agg0, agg1, y1p, dis, b1, w2)


def _mid2_body(a0, a1, y, dis, b, w, out):
    dis_v = dis[...]
    h = jnp.maximum((a0[...] + a1[...] + y[...]) * dis_v + b[...], 0.0)
    out[...] = jnp.dot(h, w[...][0],
                       preferred_element_type=jnp.float32) * dis_v


def _tc_mid2(agg0, agg1, y2p, dis, b2, w3r):
    return pl.pallas_call(
        _mid2_body,
        grid=(_NB, 2),
        in_specs=[
            pl.BlockSpec((_TN, 128), lambda i, c: (i, 0)),
            pl.BlockSpec((_TN, 128), lambda i, c: (i, 0)),
            pl.BlockSpec((_TN, 128), lambda i, c: (i, 0)),
            pl.BlockSpec((_TN, 1), lambda i, c: (i, 0)),
            pl.BlockSpec((1, 128), lambda i, c: (0, 0)),
            pl.BlockSpec((1, 128, 128), lambda i, c: (c, 0, 0)),
        ],
        out_specs=pl.BlockSpec((_TN, 128), lambda i, c: (c * _NB + i, 0)),
        out_shape=jax.ShapeDtypeStruct((2 * _N, 128), jnp.float32),
    )(agg0, agg1, y2p, dis, b2, w3r)


def _pool_body(a0, a1, y0, y1, dis, b, batch, psum, pcnt):
    i = pl.program_id(0)
    pre = jnp.concatenate([a0[...] + y0[...], a1[...] + y1[...]], axis=1)
    h = jnp.maximum(pre * dis[...] + b[...], 0.0)
    oh = (batch[...] == lax.broadcasted_iota(jnp.int32, (_TN, _B), 1))
    oh = oh.astype(jnp.float32)

    @pl.when(i == 0)
    def _():
        psum[...] = jnp.zeros_like(psum)
        pcnt[...] = jnp.zeros_like(pcnt)

    dn = (((0,), (0,)), ((), ()))
    psum[...] += lax.dot_general(oh, h, dn,
                                 preferred_element_type=jnp.float32)
    pcnt[...] += lax.dot_general(oh, jnp.ones((_TN, 128), jnp.float32), dn,
                                 preferred_element_type=jnp.float32)


def _tc_pool(agg0, agg1, y3, dis, b, batch2):
    return pl.pallas_call(
        _pool_body,
        grid=(_NB,),
        in_specs=[
            pl.BlockSpec((_TN, 128), lambda i: (i, 0)),
            pl.BlockSpec((_TN, 128), lambda i: (i, 0)),
            pl.BlockSpec((_TN, 128), lambda i: (i, 0)),
            pl.BlockSpec((_TN, 128), lambda i: (_NB + i, 0)),
            pl.BlockSpec((_TN, 1), lambda i: (i, 0)),
            pl.BlockSpec((1, 256), lambda i: (0, 0)),
            pl.BlockSpec((_TN, 1), lambda i: (i, 0)),
        ],
        out_specs=[
            pl.BlockSpec((_B, 256), lambda i: (0, 0)),
            pl.BlockSpec((_B, 128), lambda i: (0, 0)),
        ],
        out_shape=[
            jax.ShapeDtypeStruct((_B, 256), jnp.float32),
            jax.ShapeDtypeStruct((_B, 128), jnp.float32),
        ],
    )(agg0, agg1, y3, y3, dis, b, batch2)


def _sig(x):
    return 0.5 * (jnp.tanh(0.5 * x) + 1.0)


def _head_body(psum, pcnt, wf, bif, bhf, wr, bir, bhr, wfc, bfc, out):
    cnt = jnp.maximum(pcnt[:, 0:1], 1.0)
    pooled = psum[...] / cnt

    def cell(wt, bi, bh):
        g = jnp.dot(pooled, wt[...],
                    preferred_element_type=jnp.float32) + bi[...] + bh[...]
        ii = _sig(g[:, 0:128])
        gg = jnp.tanh(g[:, 256:384])
        oo = _sig(g[:, 384:512])
        return oo * jnp.tanh(ii * gg)

    hf = cell(wf, bif, bhf)
    hr = cell(wr, bir, bhr)
    last = jnp.concatenate([hf, hr], axis=1)
    logits = jnp.dot(last, wfc[...],
                     preferred_element_type=jnp.float32) + bfc[...]
    m = jnp.max(logits, axis=1, keepdims=True)
    z = logits - m
    out[...] = z - jnp.log(jnp.sum(jnp.exp(z), axis=1, keepdims=True))


def _tc_head(psum, pcnt, wf, bif, bhf, wr, bir, bhr, wfc, bfc):
    return pl.pallas_call(
        _head_body,
        out_shape=jax.ShapeDtypeStruct((_B, 500), jnp.float32),
    )(psum, pcnt, wf, bif, bhf, wr, bir, bhr, wfc, bfc)


def kernel(x, edge_index, batch, W1, b1, W2, b2, W3, b3,
           Wih_f, Whh_f, bih_f, bhh_f, Wih_r, Whh_r, bih_r, bhh_r,
           Wfc, bfc):
    src = edge_index[0].astype(jnp.int32)
    dst = edge_index[1].astype(jnp.int32)
    pad = _EP - src.shape[0]
    pad_i = jnp.arange(pad, dtype=jnp.int32)
    srcp = jnp.concatenate([src, pad_i % _N])
    dstp = jnp.concatenate([dst, _N + pad_i % (_NACC - _N)])
    src_e = srcp.reshape(_CH, _K)
    dst_e = dstp.reshape(_CH, _K)
    src_f = jnp.concatenate([srcp, srcp + _N]).reshape(2 * _CH, _K)
    dst_f = jnp.concatenate([dst_e, dst_e])
    batch2 = batch.astype(jnp.int32).reshape(_N, 1)

    deg0, deg1 = _sc_degree(dst_e)
    y1p, dis = _tc_layer0(deg0, deg1, x, W1)
    agg1a, agg1b = _sc_scatter(y1p, src_e, dst_e, _SPLIT0, _CH - _SPLIT0)
    y2p = _tc_mid1(agg1a, agg1b, y1p, dis, b1.reshape(1, -1), W2)
    agg2a, agg2b = _sc_scatter(y2p, src_e, dst_e, _SPLIT0, _CH - _SPLIT0)
    w3r = W3.reshape(128, 2, 128).transpose(1, 0, 2)
    y3 = _tc_mid2(agg2a, agg2b, y2p, dis, b2.reshape(1, -1), w3r)
    agg3a, agg3b = _sc_scatter(y3, src_f, dst_f, _CH, _CH)
    psum, pcnt = _tc_pool(agg3a, agg3b, y3, dis, b3.reshape(1, -1), batch2)
    return _tc_head(psum, pcnt,
                    Wih_f.T, bih_f.reshape(1, -1), bhh_f.reshape(1, -1),
                    Wih_r.T, bih_r.reshape(1, -1), bhh_r.reshape(1, -1),
                    Wfc.T, bfc.reshape(1, -1))


# speedup vs baseline: 2.9928x; 1.0631x over previous
import functools

import jax
import jax.numpy as jnp
from jax import lax
from jax.experimental import pallas as pl
from jax.experimental.pallas import tpu as pltpu
from jax.experimental.pallas import tpu_sc as plsc

_N = 10000
_B = 16
_NC = 2
_NS = 16
_K = 128
_G = 40
_CHW = 80
_CH = _CHW * _NC * _NS
_EP = _CH * _K
_NACC = 10240
_RT = _NACC // _NS
_ZR = 64
_TN = 2000
_NB = _N // _TN
_SPLIT0 = 1280


def _sc_mesh():
    return plsc.VectorSubcoreMesh(
        core_axis_name="c", subcore_axis_name="s",
        num_cores=_NC, num_subcores=_NS)


def _sc_degree(dst2):
    @functools.partial(
        pl.kernel,
        out_type=[jax.ShapeDtypeStruct((_NACC, 16), jnp.float32),
                  jax.ShapeDtypeStruct((_NACC, 16), jnp.float32)],
        mesh=_sc_mesh(),
        scratch_types=[
            pltpu.VMEM((_G, _K), jnp.int32),
            pltpu.VMEM((_K, 16), jnp.float32),
            pltpu.VMEM((_ZR, 16), jnp.float32),
            pltpu.VMEM_SHARED((_NACC, 16), jnp.float32),
        ],
    )
    def deg_k(dst_hbm, out0_hbm, out1_hbm, dst_g, ones_v, zb, acc):
        c = lax.axis_index("c")
        s = lax.axis_index("s")
        w = s * _NC + c

        def fillz(r, carry):
            zb[r, pl.ds(0, 16)] = jnp.zeros((16,), jnp.float32)
            return carry
        lax.fori_loop(0, _ZR, fillz, 0)

        def fillo(r, carry):
            ones_v[r, pl.ds(0, 16)] = jnp.ones((16,), jnp.float32)
            return carry
        lax.fori_loop(0, _K, fillo, 0)

        def zcopy(q, carry):
            pltpu.sync_copy(zb, acc.at[pl.ds(s * _RT + q * _ZR, _ZR)])
            return carry
        lax.fori_loop(0, _RT // _ZR, zcopy, 0)
        plsc.subcore_barrier()

        def group(g, carry):
            pltpu.sync_copy(dst_hbm.at[pl.ds(w * _CHW + g * _G, _G)], dst_g)

            def body(j, carry2):
                pltpu.sync_copy(ones_v, acc.at[dst_g.at[j]], add=True)
                return carry2
            lax.fori_loop(0, _G, body, 0)
            return carry
        lax.fori_loop(0, _CHW // _G, group, 0)

        plsc.subcore_barrier()

        @pl.when(c == 0)
        def _():
            pltpu.sync_copy(acc.at[pl.ds(s * _RT, _RT)],
                            out0_hbm.at[pl.ds(s * _RT, _RT)])

        @pl.when(c == 1)
        def _():
            pltpu.sync_copy(acc.at[pl.ds(s * _RT, _RT)],
                            out1_hbm.at[pl.ds(s * _RT, _RT)])

    return deg_k(dst2)


def _sc_scatter(y2, src_t, dst_t, cps0, cps1):
    cht0 = cps0 // _NS
    cht1 = cps1 // _NS

    @functools.partial(
        pl.kernel,
        out_type=[jax.ShapeDtypeStruct((_NACC, 128), jnp.float32),
                  jax.ShapeDtypeStruct((_NACC, 128), jnp.float32)],
        mesh=_sc_mesh(),
        scratch_types=[
            pltpu.VMEM((_G, _K), jnp.int32),
            pltpu.VMEM((_G, _K), jnp.int32),
            pltpu.VMEM((_K, 128), jnp.float32),
            pltpu.VMEM((_ZR, 128), jnp.float32),
            pltpu.SemaphoreType.DMA,
            pltpu.VMEM_SHARED((_NACC, 128), jnp.float32),
        ],
    )
    def scat_k(y_hbm, src_hbm, dst_hbm, out0_hbm, out1_hbm,
               src_g, dst_g, rows, zb, sem, acc):
        c = lax.axis_index("c")
        s = lax.axis_index("s")

        def fillz(r, carry):
            for q in range(8):
                zb[r, pl.ds(q * 16, 16)] = jnp.zeros((16,), jnp.float32)
            return carry
        lax.fori_loop(0, _ZR, fillz, 0)

        def zcopy(q, carry):
            pltpu.sync_copy(zb, acc.at[pl.ds(s * _RT + q * _ZR, _ZR)])
            return carry
        lax.fori_loop(0, _RT // _ZR, zcopy, 0)
        plsc.subcore_barrier()

        tile_base = jnp.where(c == 0, s * cht0, cps0 + s * cht1)
        groups = jnp.where(c == 0, cht0 // _G, cht1 // _G)

        def group(g, carry):
            base = tile_base + g * _G
            pltpu.sync_copy(src_hbm.at[pl.ds(base, _G)], src_g)
            pltpu.sync_copy(dst_hbm.at[pl.ds(base, _G)], dst_g)

            def body(j, carry2):
                pltpu.async_copy(y_hbm.at[src_g.at[j]], rows, sem).wait()
                pltpu.sync_copy(rows, acc.at[dst_g.at[j]], add=True)
                return carry2
            lax.fori_loop(0, _G, body, 0)
            return carry
        lax.fori_loop(0, groups, group, 0)

        plsc.subcore_barrier()

        @pl.when(c == 0)
        def _():
            pltpu.sync_copy(acc.at[pl.ds(s * _RT, _RT)],
                            out0_hbm.at[pl.ds(s * _RT, _RT)])

        @pl.when(c == 1)
        def _():
            pltpu.sync_copy(acc.at[pl.ds(s * _RT, _RT)],
                            out1_hbm.at[pl.ds(s * _RT, _RT)])

    return scat_k(y2, src_t, dst_t)


def _layer0_body(d0, d1, x, w, y_out, dis_out):
    deg = d0[:, 0:1] + d1[:, 0:1] + 1.0
    dis = lax.rsqrt(deg)
    y = jnp.dot(x[...], w[...], preferred_element_type=jnp.float32) * dis
    y_out[...] = jnp.concatenate(
        [y, jnp.zeros((_TN, 64), jnp.float32)], axis=1)
    dis_out[...] = dis


def _tc_layer0(deg0, deg1, x, w1):
    return pl.pallas_call(
        _layer0_body,
        grid=(_NB,),
        in_specs=[
            pl.BlockSpec((_TN, 16), lambda i: (i, 0)),
            pl.BlockSpec((_TN, 16), lambda i: (i, 0)),
            pl.BlockSpec((_TN, 128), lambda i: (i, 0)),
            pl.BlockSpec((128, 64), lambda i: (0, 0)),
        ],
        out_specs=[
            pl.BlockSpec((_TN, 128), lambda i: (i, 0)),
            pl.BlockSpec((_TN, 1), lambda i: (i, 0)),
        ],
        out_shape=[
            jax.ShapeDtypeStruct((_N, 128), jnp.float32),
            jax.ShapeDtypeStruct((_N, 1), jnp.float32),
        ],
    )(deg0, deg1, x, w1)


def _mid1_body(a0, a1, y, dis, b, w, out):
    pre = (a0[...] + a1[...] + y[...])[:, 0:64]
    dis_v = dis[...]
    h = jnp.maximum(pre * dis_v + b[...], 0.0)
    out[...] = jnp.dot(h, w[...], preferred_element_type=jnp.float32) * dis_v


def _tc_mid1(agg0, agg1, y1p, dis, b1, w2):
    return pl.pallas_call(
        _mid1_body,
        grid=(_NB,),
        in_specs=[
            pl.BlockSpec((_TN, 128), lambda i: (i, 0)),
            pl.BlockSpec((_TN, 128), lambda i: (i, 0)),
            pl.BlockSpec((_TN, 128), lambda i: (i, 0)),
            pl.BlockSpec((_TN, 1), lambda i: (i, 0)),
            pl.BlockSpec((1, 64), lambda i: (0, 0)),
            pl.BlockSpec((64, 128), lambda i: (0, 0)),
        ],
        out_specs=pl.BlockSpec((_TN, 128), lambda i: (i, 0)),
        out_shape=jax.ShapeDtypeStruct((_N, 128), jnp.float32),
    )(agg0, agg1, y1p, dis, b1, w2)


def _mid2_body(a0, a1, y, dis, b, w, outa, outb):
    dis_v = dis[...]
    h = jnp.maximum((a0[...] + a1[...] + y[...]) * dis_v + b[...], 0.0)
    wv = w[...]
    outa[...] = jnp.dot(h, wv[:, 0:128],
                        preferred_element_type=jnp.float32) * dis_v
    outb[...] = jnp.dot(h, wv[:, 128:256],
                        preferred_element_type=jnp.float32) * dis_v


def _tc_mid2(agg0, agg1, y2p, dis, b2, w3):
    return pl.pallas_call(
        _mid2_body,
        grid=(_NB,),
        in_specs=[
            pl.BlockSpec((_TN, 128), lambda i: (i, 0)),
            pl.BlockSpec((_TN, 128), lambda i: (i, 0)),
            pl.BlockSpec((_TN, 128), lambda i: (i, 0)),
            pl.BlockSpec((_TN, 1), lambda i: (i, 0)),
            pl.BlockSpec((1, 128), lambda i: (0, 0)),
            pl.BlockSpec((128, 256), lambda i: (0, 0)),
        ],
        out_specs=[
            pl.BlockSpec((_TN, 128), lambda i: (i, 0)),
            pl.BlockSpec((_TN, 128), lambda i: (i, 0)),
        ],
        out_shape=[
            jax.ShapeDtypeStruct((_N, 128), jnp.float32),
            jax.ShapeDtypeStruct((_N, 128), jnp.float32),
        ],
    )(agg0, agg1, y2p, dis, b2, w3)


def _pool_body(a0, a1, y0, y1, dis, b, batch, psum, pcnt):
    i = pl.program_id(0)
    pre = jnp.concatenate([a0[...] + y0[...], a1[...] + y1[...]], axis=1)
    h = jnp.maximum(pre * dis[...] + b[...], 0.0)
    oh = (batch[...] == lax.broadcasted_iota(jnp.int32, (_TN, _B), 1))
    oh = oh.astype(jnp.float32)

    @pl.when(i == 0)
    def _():
        psum[...] = jnp.zeros_like(psum)
        pcnt[...] = jnp.zeros_like(pcnt)

    dn = (((0,), (0,)), ((), ()))
    psum[...] += lax.dot_general(oh, h, dn,
                                 preferred_element_type=jnp.float32)
    pcnt[...] += lax.dot_general(oh, jnp.ones((_TN, 128), jnp.float32), dn,
                                 preferred_element_type=jnp.float32)


def _tc_pool(agg0, agg1, y3a, y3b, dis, b, batch2):
    return pl.pallas_call(
        _pool_body,
        grid=(_NB,),
        in_specs=[
            pl.BlockSpec((_TN, 128), lambda i: (i, 0)),
            pl.BlockSpec((_TN, 128), lambda i: (i, 0)),
            pl.BlockSpec((_TN, 128), lambda i: (i, 0)),
            pl.BlockSpec((_TN, 128), lambda i: (i, 0)),
            pl.BlockSpec((_TN, 1), lambda i: (i, 0)),
            pl.BlockSpec((1, 256), lambda i: (0, 0)),
            pl.BlockSpec((_TN, 1), lambda i: (i, 0)),
        ],
        out_specs=[
            pl.BlockSpec((_B, 256), lambda i: (0, 0)),
            pl.BlockSpec((_B, 128), lambda i: (0, 0)),
        ],
        out_shape=[
            jax.ShapeDtypeStruct((_B, 256), jnp.float32),
            jax.ShapeDtypeStruct((_B, 128), jnp.float32),
        ],
    )(agg0, agg1, y3a, y3b, dis, b, batch2)


def _sig(x):
    return 0.5 * (jnp.tanh(0.5 * x) + 1.0)


def _head_body(psum, pcnt, wf, bif, bhf, wr, bir, bhr, wfc, bfc, out):
    cnt = jnp.maximum(pcnt[:, 0:1], 1.0)
    pooled = psum[...] / cnt

    def cell(wt, bi, bh):
        g = jnp.dot(pooled, wt[...],
                    preferred_element_type=jnp.float32) + bi[...] + bh[...]
        ii = _sig(g[:, 0:128])
        gg = jnp.tanh(g[:, 256:384])
        oo = _sig(g[:, 384:512])
        return oo * jnp.tanh(ii * gg)

    hf = cell(wf, bif, bhf)
    hr = cell(wr, bir, bhr)
    last = jnp.concatenate([hf, hr], axis=1)
    logits = jnp.dot(last, wfc[...],
                     preferred_element_type=jnp.float32) + bfc[...]
    m = jnp.max(logits, axis=1, keepdims=True)
    z = logits - m
    out[...] = z - jnp.log(jnp.sum(jnp.exp(z), axis=1, keepdims=True))


def _tc_head(psum, pcnt, wf, bif, bhf, wr, bir, bhr, wfc, bfc):
    return pl.pallas_call(
        _head_body,
        out_shape=jax.ShapeDtypeStruct((_B, 500), jnp.float32),
    )(psum, pcnt, wf, bif, bhf, wr, bir, bhr, wfc, bfc)


def kernel(x, edge_index, batch, W1, b1, W2, b2, W3, b3,
           Wih_f, Whh_f, bih_f, bhh_f, Wih_r, Whh_r, bih_r, bhh_r,
           Wfc, bfc):
    src = edge_index[0].astype(jnp.int32)
    dst = edge_index[1].astype(jnp.int32)
    pad = _EP - src.shape[0]
    pad_i = jnp.arange(pad, dtype=jnp.int32)
    srcp = jnp.concatenate([src, pad_i % _N])
    dstp = jnp.concatenate([dst, _N + pad_i % (_NACC - _N)])
    src_e = srcp.reshape(_CH, _K)
    dst_e = dstp.reshape(_CH, _K)
    src_f = jnp.concatenate([srcp, srcp + _N]).reshape(2 * _CH, _K)
    dst_f = jnp.concatenate([dst_e, dst_e])
    batch2 = batch.astype(jnp.int32).reshape(_N, 1)

    deg0, deg1 = _sc_degree(dst_e)
    y1p, dis = _tc_layer0(deg0, deg1, x, W1)
    agg1a, agg1b = _sc_scatter(y1p, src_e, dst_e, _SPLIT0, _CH - _SPLIT0)
    y2p = _tc_mid1(agg1a, agg1b, y1p, dis, b1.reshape(1, -1), W2)
    agg2a, agg2b = _sc_scatter(y2p, src_e, dst_e, _SPLIT0, _CH - _SPLIT0)
    y3a, y3b = _tc_mid2(agg2a, agg2b, y2p, dis, b2.reshape(1, -1), W3)
    y3 = jnp.concatenate([y3a, y3b], axis=0)
    agg3a, agg3b = _sc_scatter(y3, src_f, dst_f, _CH, _CH)
    psum, pcnt = _tc_pool(agg3a, agg3b, y3a, y3b, dis, b3.reshape(1, -1),
                          batch2)
    return _tc_head(psum, pcnt,
                    Wih_f.T, bih_f.reshape(1, -1), bhh_f.reshape(1, -1),
                    Wih_r.T, bih_r.reshape(1, -1), bhh_r.reshape(1, -1),
                    Wfc.T, bfc.reshape(1, -1))
